# baseline (speedup 1.0000x reference)
import functools

import jax
import jax.numpy as jnp
from jax import lax
from jax.experimental import pallas as pl
from jax.experimental.pallas import tpu as pltpu
from jax.experimental.pallas import tpu_sc as plsc


def _round_up(a: int, b: int) -> int:
    return (a + b - 1) // b * b


def kernel(x, edge_index, W_emb, W_rel, b_rel, W_root, W_sum, b_sum, W_out, b_out):
    f32 = jnp.float32
    N, D_IN = x.shape
    E = edge_index.shape[1]
    DEPTH, D_H, _ = W_rel.shape
    K1 = DEPTH + 1

    NC, NS, LN = 2, 16, 128
    NW = NC * NS
    CR = 16
    ROWS = _round_up(-(-E // LN), NW * CR)
    EPAD = ROWS * LN
    RW = ROWS // NW
    NCH = RW // CR
    NPAD = _round_up(N, NS * 8)
    ZC = NPAD // NS

    src = edge_index[0]
    dst = edge_index[1]
    pad = EPAD - E
    srcp = jnp.concatenate([src, jnp.full((pad,), N, jnp.int32)]).reshape(ROWS, LN)
    dstp = jnp.concatenate([dst, jnp.zeros((pad,), jnp.int32)]).reshape(ROWS, LN)

    mesh = plsc.VectorSubcoreMesh(core_axis_name="c", subcore_axis_name="s")

    @functools.partial(
        pl.kernel,
        out_type=jax.ShapeDtypeStruct((NC * NPAD,), f32),
        mesh=mesh,
        scratch_types=[
            pltpu.VMEM((RW, LN), jnp.int32),
            pltpu.VMEM((RW, LN), f32),
            pltpu.VMEM((ZC,), f32),
            pltpu.VMEM_SHARED((NPAD,), f32),
            pltpu.SemaphoreType.DMA,
        ],
    )
    def spmv(src_h, dst_h, v_h, out_h, idx_v, val_v, stage_v, acc_sh, sem):
        cid = lax.axis_index("c")
        sid = lax.axis_index("s")
        wid = sid * NC + cid
        off = pl.multiple_of(sid * ZC, 8)
        sl = pl.ds(off, ZC)

        zv = jnp.zeros((16,), f32)

        def zbody(i, carry):
            stage_v[pl.ds(pl.multiple_of(i * 16, 16), 16)] = zv
            return carry

        lax.fori_loop(0, ZC // 16, zbody, 0)
        pltpu.sync_copy(stage_v, acc_sh.at[sl])
        plsc.subcore_barrier()

        base = pl.multiple_of(wid * RW, 8)

        pltpu.sync_copy(dst_h.at[pl.ds(base, RW)], idx_v)

        def drain_gather():
            pltpu.make_async_copy(v_h.at[pl.ds(0, LN)], val_v.at[0], sem).wait()

        def g_body(t, carry):
            r0 = pl.multiple_of(t * CR, 8)

            @pl.when(t > 0)
            def _():
                for _j in range(CR):
                    drain_gather()

            for j in range(CR):
                pltpu.async_copy(v_h.at[idx_v.at[r0 + j]], val_v.at[r0 + j],
                                 sem)
            return carry

        lax.fori_loop(0, NCH, g_body, 0)
        for _j in range(CR):
            drain_gather()

        pltpu.sync_copy(src_h.at[pl.ds(base, RW)], idx_v)

        def drain_scatter():
            pltpu.make_async_copy(val_v.at[0], acc_sh.at[pl.ds(0, LN)],
                                  sem).wait()

        def s_body(t, carry):
            r0 = pl.multiple_of(t * CR, 8)

            @pl.when(t > 0)
            def _():
                for _j in range(CR):
                    drain_scatter()

            for j in range(CR):
                pltpu.async_copy(val_v.at[r0 + j], acc_sh.at[idx_v.at[r0 + j]],
                                 sem, add=True)
            return carry

        lax.fori_loop(0, NCH, s_body, 0)
        for _j in range(CR):
            drain_scatter()

        plsc.subcore_barrier()
        oout = pl.multiple_of(cid * NPAD + off, 8)
        pltpu.sync_copy(acc_sh.at[sl], stage_v)
        pltpu.sync_copy(stage_v, out_h.at[pl.ds(oout, ZC)])

    v = jnp.ones((N,), f32)
    vs = [v]
    for _ in range(DEPTH):
        parts = spmv(srcp, dstp, v).reshape(NC, NPAD)
        v = parts[0, :N] + parts[1, :N]
        vs.append(v)

    bf16 = jnp.bfloat16
    GP = 16
    NPTC = _round_up(N, GP * 128)
    BN = NPTC // GP
    KP = _round_up(K1, 16)
    DA = _round_up(D_IN + 1, 32)
    Vm = jnp.stack(vs)
    Vp = jnp.zeros((KP, NPTC), f32).at[:K1, :N].set(Vm)
    xp = (jnp.zeros((NPTC, DA), f32)
          .at[:N, :D_IN].set(x)).astype(bf16)
    Wemb_p = jnp.zeros((DA, D_H), f32).at[:D_IN].set(W_emb).astype(bf16)

    def pbody(v_ref, x_ref, we_ref, u_ref, s_ref):
        @pl.when(pl.program_id(0) == 0)
        def _init():
            u_ref[...] = jnp.zeros_like(u_ref)
            s_ref[...] = jnp.zeros_like(s_ref)
        vblk = v_ref[...]
        h0 = jnp.dot(x_ref[...], we_ref[...], preferred_element_type=f32)
        u_ref[...] += jnp.dot(vblk, h0, preferred_element_type=f32,
                              precision=lax.Precision.HIGHEST)
        s_ref[...] += jnp.sum(vblk, axis=1, keepdims=True)

    U16, S16 = pl.pallas_call(
        pbody,
        grid=(GP,),
        in_specs=[pl.BlockSpec((KP, BN), lambda g: (0, g)),
                  pl.BlockSpec((BN, DA), lambda g: (g, 0)),
                  pl.BlockSpec((DA, D_H), lambda g: (0, 0))],
        out_specs=[pl.BlockSpec((KP, D_H), lambda g: (0, 0)),
                   pl.BlockSpec((KP, 128), lambda g: (0, 0))],
        out_shape=[jax.ShapeDtypeStruct((KP, D_H), f32),
                   jax.ShapeDtypeStruct((KP, 128), f32)],
    )(Vp, xp, Wemb_p)

    brel_r = b_rel.reshape(DEPTH, 1, D_H)
    Wout_r = W_out.reshape(DEPTH, D_H, D_IN)
    bout_r = b_out.reshape(1, D_IN)

    def cbody(u_ref, s_ref, wr_ref, br_ref, wp_ref, wo_ref, bo_ref, o_ref):
        U = u_ref[...]
        sig = s_ref[...][:, :1]
        rows = lax.broadcasted_iota(jnp.int32, (KP * KP, 1), 0)
        dmask = ((rows // KP) == (rows % KP)).astype(f32)
        Y = jnp.concatenate([U[None], jnp.zeros((KP - 1, KP, D_H), f32)], 0)
        C = jnp.zeros((KP, D_H), f32)
        res = jnp.zeros((1, D_IN), f32)
        zY = jnp.zeros((1, KP, D_H), f32)
        zC = jnp.zeros((1, D_H), f32)
        zB = jnp.zeros((KP - 1, D_H), f32)
        hi = lax.Precision.HIGHEST
        bf = jnp.bfloat16
        for i in range(DEPTH):
            R = wr_ref[i].astype(bf).astype(f32)
            Pm = wp_ref[i].astype(bf).astype(f32)
            Ysh = jnp.concatenate([zY, Y[:-1]], 0).reshape(KP * KP, D_H)
            Yfl = Y.reshape(KP * KP, D_H)
            Yn = (jnp.dot(Ysh, R, preferred_element_type=f32, precision=hi)
                  + jnp.dot(Yfl, Pm, preferred_element_type=f32, precision=hi))
            Y = Yn.reshape(KP, KP, D_H)
            Csh = jnp.concatenate([zC, C[:-1]], 0)
            C = (jnp.dot(Csh, R, preferred_element_type=f32, precision=hi)
                 + jnp.dot(C, Pm, preferred_element_type=f32, precision=hi)
                 + jnp.concatenate([br_ref[i], zB], 0))
            s = (jnp.sum(Yn * dmask, axis=0, keepdims=True)
                 + jnp.sum(C * sig, axis=0, keepdims=True))
            res = res + jnp.dot(s.astype(bf).astype(f32),
                                wo_ref[i].astype(bf).astype(f32),
                                preferred_element_type=f32, precision=hi)
        o_ref[...] = res + bo_ref[...]

    res = pl.pallas_call(
        cbody,
        out_shape=jax.ShapeDtypeStruct((1, D_IN), f32),
    )(U16, S16, W_rel, brel_r, W_root, Wout_r, bout_r)
    return res.reshape(D_IN)


# speedup vs baseline: 2.4033x; 2.4033x over previous
import functools

import jax
import jax.numpy as jnp
from jax import lax
from jax.experimental import pallas as pl
from jax.experimental.pallas import tpu as pltpu
from jax.experimental.pallas import tpu_sc as plsc


def _round_up(a: int, b: int) -> int:
    return (a + b - 1) // b * b


def kernel(x, edge_index, W_emb, W_rel, b_rel, W_root, W_sum, b_sum, W_out, b_out):
    f32 = jnp.float32
    N, D_IN = x.shape
    E = edge_index.shape[1]
    DEPTH, D_H, _ = W_rel.shape
    K1 = DEPTH + 1

    NC, NS, LN = 2, 16, 128
    NW = NC * NS
    CR = 8
    ROWS = _round_up(-(-E // LN), NW * CR)
    EPAD = ROWS * LN
    RW = ROWS // NW
    NCH = RW // CR
    NPAD = _round_up(N, NS * 8)
    ZC = NPAD // NS

    src = edge_index[0]
    dst = edge_index[1]
    pad = EPAD - E
    srcp = jnp.concatenate([src, jnp.full((pad,), N, jnp.int32)]
                           ).reshape(NW, RW, LN)
    dstp = jnp.concatenate([dst, jnp.zeros((pad,), jnp.int32)]
                           ).reshape(NW, RW, LN)

    mesh = plsc.VectorSubcoreMesh(core_axis_name="c", subcore_axis_name="s")

    @functools.partial(
        pl.kernel,
        out_type=jax.ShapeDtypeStruct((NC * NPAD,), f32),
        mesh=mesh,
        scratch_types=[
            pltpu.VMEM((RW, LN), jnp.int32),
            pltpu.VMEM((RW, LN), jnp.int32),
            pltpu.VMEM((4 * CR, LN), f32),
            pltpu.VMEM((ZC,), f32),
            pltpu.VMEM_SHARED((NPAD,), f32),
            pltpu.SemaphoreType.DMA,
            pltpu.SemaphoreType.DMA,
            pltpu.SemaphoreType.DMA,
            pltpu.SemaphoreType.DMA,
        ],
    )
    def spmv(src_h, dst_h, v_h, out_h, idxd_v, idxs_v, val_v, stage_v, acc_sh,
             semg, sems, semid, semis):
        cid = lax.axis_index("c")
        sid = lax.axis_index("s")
        wid = sid * NC + cid
        off = pl.multiple_of(sid * ZC, 8)
        sl = pl.ds(off, ZC)
        CRL = CR * LN

        pltpu.async_copy(dst_h.at[wid], idxd_v, semid)
        pltpu.async_copy(src_h.at[wid], idxs_v, semis)

        zv = jnp.zeros((16,), f32)

        def zbody(i, carry):
            stage_v[pl.ds(pl.multiple_of(i * 16, 16), 16)] = zv
            return carry

        lax.fori_loop(0, ZC // 16, zbody, 0)
        pltpu.sync_copy(stage_v, acc_sh.at[sl])
        plsc.subcore_barrier()

        def wait_units(sem, kind):
            if kind == "g":
                pltpu.make_async_copy(v_h.at[pl.ds(0, CRL)],
                                      stage_v.at[pl.ds(0, CRL)], sem).wait()
            elif kind == "s":
                pltpu.make_async_copy(stage_v.at[pl.ds(0, CRL)],
                                      acc_sh.at[pl.ds(0, CRL)], sem).wait()
            else:
                pltpu.make_async_copy(dst_h.at[wid, pl.ds(0, CR)],
                                      idxd_v.at[pl.ds(0, CR)], sem).wait()

        def body(t, carry):
            @pl.when(t >= 2)
            def _():
                wait_units(semg, "g")

            @pl.when(t >= 4)
            def _():
                wait_units(sems, "s")

            @pl.when(t >= 2)
            def _():
                wait_units(semis, "i")
                b = t - 2
                r0 = pl.multiple_of(b * CR, 8)
                v0 = pl.multiple_of((b % 4) * CR, 8)
                for j in range(CR):
                    pltpu.async_copy(val_v.at[v0 + j],
                                     acc_sh.at[idxs_v.at[r0 + j]], sems,
                                     add=True)

            @pl.when(t < NCH)
            def _():
                wait_units(semid, "i")
                r0 = pl.multiple_of(t * CR, 8)
                v0 = pl.multiple_of((t % 4) * CR, 8)
                for j in range(CR):
                    pltpu.async_copy(v_h.at[idxd_v.at[r0 + j]],
                                     val_v.at[v0 + j], semg)
            return carry

        lax.fori_loop(0, NCH + 2, body, 0)
        wait_units(sems, "s")
        wait_units(sems, "s")

        plsc.subcore_barrier()
        oout = pl.multiple_of(cid * NPAD + off, 8)
        pltpu.sync_copy(acc_sh.at[sl], stage_v)
        pltpu.sync_copy(stage_v, out_h.at[pl.ds(oout, ZC)])

    v = jnp.ones((N,), f32)
    vs = [v]
    for _ in range(DEPTH):
        parts = spmv(srcp, dstp, v).reshape(NC, NPAD)
        v = parts[0, :N] + parts[1, :N]
        vs.append(v)

    bf16 = jnp.bfloat16
    GP = 16
    NPTC = _round_up(N, GP * 128)
    BN = NPTC // GP
    KP = _round_up(K1, 16)
    DA = _round_up(D_IN + 1, 32)
    Vm = jnp.stack(vs)
    Vp = jnp.zeros((KP, NPTC), f32).at[:K1, :N].set(Vm)
    xp = (jnp.zeros((NPTC, DA), f32)
          .at[:N, :D_IN].set(x)).astype(bf16)
    Wemb_p = jnp.zeros((DA, D_H), f32).at[:D_IN].set(W_emb).astype(bf16)

    def pbody(v_ref, x_ref, we_ref, u_ref, s_ref):
        @pl.when(pl.program_id(0) == 0)
        def _init():
            u_ref[...] = jnp.zeros_like(u_ref)
            s_ref[...] = jnp.zeros_like(s_ref)
        vblk = v_ref[...]
        h0 = jnp.dot(x_ref[...], we_ref[...], preferred_element_type=f32)
        u_ref[...] += jnp.dot(vblk, h0, preferred_element_type=f32,
                              precision=lax.Precision.HIGHEST)
        s_ref[...] += jnp.sum(vblk, axis=1, keepdims=True)

    U16, S16 = pl.pallas_call(
        pbody,
        grid=(GP,),
        in_specs=[pl.BlockSpec((KP, BN), lambda g: (0, g)),
                  pl.BlockSpec((BN, DA), lambda g: (g, 0)),
                  pl.BlockSpec((DA, D_H), lambda g: (0, 0))],
        out_specs=[pl.BlockSpec((KP, D_H), lambda g: (0, 0)),
                   pl.BlockSpec((KP, 128), lambda g: (0, 0))],
        out_shape=[jax.ShapeDtypeStruct((KP, D_H), f32),
                   jax.ShapeDtypeStruct((KP, 128), f32)],
    )(Vp, xp, Wemb_p)

    brel_r = b_rel.reshape(DEPTH, 1, D_H)
    Wout_r = W_out.reshape(DEPTH, D_H, D_IN)
    bout_r = b_out.reshape(1, D_IN)

    def cbody(u_ref, s_ref, wr_ref, br_ref, wp_ref, wo_ref, bo_ref, o_ref):
        U = u_ref[...]
        sig = s_ref[...][:, :1]
        rows = lax.broadcasted_iota(jnp.int32, (KP * KP, 1), 0)
        dmask = ((rows // KP) == (rows % KP)).astype(f32)
        Y = jnp.concatenate([U[None], jnp.zeros((KP - 1, KP, D_H), f32)], 0)
        C = jnp.zeros((KP, D_H), f32)
        res = jnp.zeros((1, D_IN), f32)
        zY = jnp.zeros((1, KP, D_H), f32)
        zC = jnp.zeros((1, D_H), f32)
        zB = jnp.zeros((KP - 1, D_H), f32)
        hi = lax.Precision.HIGHEST
        bf = jnp.bfloat16
        for i in range(DEPTH):
            R = wr_ref[i].astype(bf).astype(f32)
            Pm = wp_ref[i].astype(bf).astype(f32)
            Ysh = jnp.concatenate([zY, Y[:-1]], 0).reshape(KP * KP, D_H)
            Yfl = Y.reshape(KP * KP, D_H)
            Yn = (jnp.dot(Ysh, R, preferred_element_type=f32, precision=hi)
                  + jnp.dot(Yfl, Pm, preferred_element_type=f32, precision=hi))
            Y = Yn.reshape(KP, KP, D_H)
            Csh = jnp.concatenate([zC, C[:-1]], 0)
            C = (jnp.dot(Csh, R, preferred_element_type=f32, precision=hi)
                 + jnp.dot(C, Pm, preferred_element_type=f32, precision=hi)
                 + jnp.concatenate([br_ref[i], zB], 0))
            s = (jnp.sum(Yn * dmask, axis=0, keepdims=True)
                 + jnp.sum(C * sig, axis=0, keepdims=True))
            res = res + jnp.dot(s.astype(bf).astype(f32),
                                wo_ref[i].astype(bf).astype(f32),
                                preferred_element_type=f32, precision=hi)
        o_ref[...] = res + bo_ref[...]

    res = pl.pallas_call(
        cbody,
        out_shape=jax.ShapeDtypeStruct((1, D_IN), f32),
    )(U16, S16, W_rel, brel_r, W_root, Wout_r, bout_r)
    return res.reshape(D_IN)
